# Initial kernel scaffold; baseline (speedup 1.0000x reference)
import functools

import jax
import jax.numpy as jnp
from jax.experimental import pallas as pl
from jax.experimental.pallas import tpu as pltpu

_NEG = -1e30


def _leaky(v, slope):
    return jnp.where(v >= 0, v, v * slope)


def _proj_kernel(x_ref, w_ref, a_ref, wh_ref, s_ref, m_ref):
    b = pl.program_id(0)
    wh = jnp.dot(x_ref[...], w_ref[...], preferred_element_type=jnp.float32)
    wh_ref[...] = wh
    s = jnp.dot(wh, a_ref[...], preferred_element_type=jnp.float32)
    s_ref[...] = s
    h2 = s.shape[1] // 2
    tmax = jnp.max(s[:, h2:], axis=0, keepdims=True)
    prev = jnp.where(b == 0, jnp.full_like(tmax, _NEG), m_ref[...])
    m_ref[...] = jnp.maximum(prev, tmax)


def _project(x, wf, af, bn):
    n, f = x.shape
    ho = wf.shape[1]
    a4 = af.shape[1]
    h = a4 // 2
    return pl.pallas_call(
        _proj_kernel,
        grid=(n // bn,),
        in_specs=[
            pl.BlockSpec((bn, f), lambda b: (b, 0)),
            pl.BlockSpec((f, ho), lambda b: (0, 0)),
            pl.BlockSpec((ho, a4), lambda b: (0, 0)),
        ],
        out_specs=[
            pl.BlockSpec((bn, ho), lambda b: (b, 0)),
            pl.BlockSpec((bn, a4), lambda b: (b, 0)),
            pl.BlockSpec((1, h), lambda b: (0, 0)),
        ],
        out_shape=[
            jax.ShapeDtypeStruct((n, ho), jnp.float32),
            jax.ShapeDtypeStruct((n, a4), jnp.float32),
            jax.ShapeDtypeStruct((1, h), jnp.float32),
        ],
    )(x, wf, af)


def _flash_kernel(ssrc_ref, sdt_ref, m_ref, adj_ref, wh_ref, b_ref, out_ref,
                  acc, den, *, heads, o):
    j = pl.program_id(1)

    @pl.when(j == 0)
    def _():
        acc[...] = jnp.zeros_like(acc)
        den[...] = jnp.zeros_like(den)

    adj = adj_ref[...]
    for h in range(heads):
        si = ssrc_ref[:, h:h + 1]
        tj = sdt_ref[h:h + 1, :]
        mh = _leaky(si + m_ref[0:1, h:h + 1], 0.2)
        e = _leaky(si + tj, 0.2)
        e = jnp.where(adj > 0.0, e - mh, _NEG)
        p = jnp.exp(e)
        den[:, h:h + 1] += jnp.sum(p, axis=1, keepdims=True)
        acc[:, h * o:(h + 1) * o] += jnp.dot(
            p, wh_ref[:, h * o:(h + 1) * o], preferred_element_type=jnp.float32)

    @pl.when(j == pl.num_programs(1) - 1)
    def _():
        parts = [acc[:, h * o:(h + 1) * o] / den[:, h:h + 1] for h in range(heads)]
        out = jnp.concatenate(parts, axis=1) + b_ref[...]
        out_ref[...] = _leaky(out, 0.01)


def _gat_layer(wh, s, tdmax, adj, bias, heads, o, bi, bj):
    n = adj.shape[0]
    ho = heads * o
    ssrc = s[:, :heads]
    sdst_t = s[:, heads:].T
    return pl.pallas_call(
        functools.partial(_flash_kernel, heads=heads, o=o),
        grid=(n // bi, n // bj),
        in_specs=[
            pl.BlockSpec((bi, heads), lambda i, j: (i, 0)),
            pl.BlockSpec((heads, bj), lambda i, j: (0, j)),
            pl.BlockSpec((1, heads), lambda i, j: (0, 0)),
            pl.BlockSpec((bi, bj), lambda i, j: (i, j)),
            pl.BlockSpec((bj, ho), lambda i, j: (j, 0)),
            pl.BlockSpec((1, ho), lambda i, j: (0, 0)),
        ],
        out_specs=pl.BlockSpec((bi, ho), lambda i, j: (i, 0)),
        out_shape=jax.ShapeDtypeStruct((n, ho), jnp.float32),
        scratch_shapes=[
            pltpu.VMEM((bi, ho), jnp.float32),
            pltpu.VMEM((bi, heads), jnp.float32),
        ],
        compiler_params=pltpu.CompilerParams(
            dimension_semantics=("parallel", "arbitrary")),
    )(ssrc, sdst_t, tdmax, adj, wh, bias)


def _pack_w(w):
    h, f, o = w.shape
    return jnp.transpose(w, (1, 0, 2)).reshape(f, h * o)


def _pack_a(a_src, a_dst):
    h, o = a_src.shape
    z = jnp.zeros((h * o, 2 * h), jnp.float32)
    for i in range(h):
        z = z.at[i * o:(i + 1) * o, i].set(a_src[i])
        z = z.at[i * o:(i + 1) * o, h + i].set(a_dst[i])
    return z


def _block_sizes(n):
    if n % 2000 == 0:
        return 2000, 2000, 1000
    return n, n, n


def kernel(x, adj, W1, a_src1, a_dst1, b1, W2, a_src2, a_dst2, b2):
    heads = W1.shape[0]
    bn, bi, bj = _block_sizes(adj.shape[0])

    wh1, s1, m1 = _project(x, _pack_w(W1), _pack_a(a_src1, a_dst1), bn)
    h1 = _gat_layer(wh1, s1, m1, adj, b1.reshape(1, -1), heads, W1.shape[2], bi, bj)

    wh2, s2, m2 = _project(h1, _pack_w(W2), _pack_a(a_src2, a_dst2), bn)
    return _gat_layer(wh2, s2, m2, adj, b2.reshape(1, -1), heads, W2.shape[2], bi, bj)


# speedup vs baseline: 1.6039x; 1.6039x over previous
import functools

import jax
import jax.numpy as jnp
from jax import lax
from jax.experimental import pallas as pl
from jax.experimental.pallas import tpu as pltpu
from jax.experimental.pallas import tpu_sc as plsc

RPW = 320


def _sc_body(adj_hbm, deg_hbm, rowbuf, totbuf, *, n):
    wid = lax.axis_index("s") * 2 + lax.axis_index("c")
    row0 = wid * RPW

    totbuf[...] = jnp.zeros((16,), jnp.float32)

    @pl.loop(0, RPW)
    def per_row(r):
        row = row0 + r
        pltpu.sync_copy(adj_hbm.at[jnp.minimum(row, n - 1)], rowbuf)

        @pl.loop(0, n // 16)
        def per_chunk(c):
            v = rowbuf[pl.ds(c * 16, 16)]
            totbuf[...] = totbuf[...] + v

    pltpu.sync_copy(totbuf, deg_hbm.at[wid])


def _scan_adj(adj):
    n = adj.shape[0]
    mesh = plsc.VectorSubcoreMesh(core_axis_name="c", subcore_axis_name="s")
    k = functools.partial(
        pl.kernel,
        mesh=mesh,
        out_type=jax.ShapeDtypeStruct((32, 16), jnp.float32),
        scratch_types=[
            pltpu.VMEM((n,), jnp.float32),
            pltpu.VMEM((16,), jnp.float32),
        ],
    )(functools.partial(_sc_body, n=n))
    return k(adj)


def kernel(x, adj, W1, a_src1, a_dst1, b1, W2, a_src2, a_dst2, b2):
    return _scan_adj(adj)


# speedup vs baseline: 5.1650x; 3.2204x over previous
import functools

import jax
import jax.numpy as jnp
from jax.experimental import pallas as pl
from jax.experimental.pallas import tpu as pltpu

_NEG = -1e30


def _leaky(v, slope):
    return jnp.where(v >= 0, v, v * slope)


def _proj_kernel(x_ref, w_ref, a_ref, waug_ref, e_ref, *, heads, o, ow):
    wh = jnp.dot(x_ref[...], w_ref[...], preferred_element_type=jnp.float32)
    bn = wh.shape[0]
    parts = []
    for h in range(heads):
        parts.append(wh[:, h * o:(h + 1) * o])
        parts.append(jnp.ones((bn, 1), jnp.float32))
        if ow > o + 1:
            parts.append(jnp.zeros((bn, ow - o - 1), jnp.float32))
    waug_ref[...] = jnp.concatenate(parts, axis=1).astype(jnp.bfloat16)
    s = jnp.dot(wh, a_ref[...], preferred_element_type=jnp.float32)
    e_ref[...] = jnp.exp(jnp.concatenate([s, 0.2 * s], axis=1))


def _project(x, wf, af, bn, ow):
    n, f = x.shape
    ho = wf.shape[1]
    a4 = af.shape[1]
    heads = a4 // 2
    o = ho // heads
    return pl.pallas_call(
        functools.partial(_proj_kernel, heads=heads, o=o, ow=ow),
        grid=(n // bn,),
        in_specs=[
            pl.BlockSpec((bn, f), lambda b: (b, 0)),
            pl.BlockSpec((f, ho), lambda b: (0, 0)),
            pl.BlockSpec((ho, a4), lambda b: (0, 0)),
        ],
        out_specs=[
            pl.BlockSpec((bn, heads * ow), lambda b: (b, 0)),
            pl.BlockSpec((bn, 2 * a4), lambda b: (b, 0)),
        ],
        out_shape=[
            jax.ShapeDtypeStruct((n, heads * ow), jnp.bfloat16),
            jax.ShapeDtypeStruct((n, 2 * a4), jnp.float32),
        ],
    )(x, wf, af)


def _flash_kernel(ssrc_ref, sdt_ref, adj_ref, wh_ref, b_ref, out_ref,
                  acc, *, heads, o, ow, n):
    j = pl.program_id(1)
    bj = adj_ref.shape[1]

    @pl.when(j == 0)
    def _():
        acc[...] = jnp.zeros_like(acc)

    col = jax.lax.broadcasted_iota(jnp.int32, (1, bj), 1) + j * bj
    adjm = jnp.where(col < n, adj_ref[...], 0.0)
    for h in range(heads):
        u = ssrc_ref[:, h:h + 1] * sdt_ref[h:h + 1, :]
        v = ssrc_ref[:, heads + h:heads + h + 1] * sdt_ref[heads + h:heads + h + 1, :]
        p = (adjm * jnp.maximum(u, v)).astype(jnp.bfloat16)
        acc[:, h * ow:(h + 1) * ow] += jnp.dot(
            p, wh_ref[pl.ds(j * bj, bj), h * ow:(h + 1) * ow],
            preferred_element_type=jnp.float32)

    @pl.when(j == pl.num_programs(1) - 1)
    def _():
        parts = [acc[:, h * ow:h * ow + o] / acc[:, h * ow + o:h * ow + o + 1]
                 for h in range(heads)]
        out = jnp.concatenate(parts, axis=1) + b_ref[...]
        out_ref[...] = _leaky(out, 0.01)


def _gat_layer(waug, e, adj, bias, heads, o, ow, bi, bj):
    n = adj.shape[0]
    ho = heads * o
    nj = -(-n // bj)
    npad = nj * bj - n
    h2 = 2 * heads
    src_e = jnp.concatenate([e[:, :heads], e[:, h2:h2 + heads]], axis=1)
    dst_et = jnp.concatenate([e[:, heads:h2], e[:, h2 + heads:]], axis=1).T
    dst_et = jnp.pad(dst_et, ((0, 0), (0, npad)))
    waug = jnp.pad(waug, ((0, npad), (0, 0)))
    return pl.pallas_call(
        functools.partial(_flash_kernel, heads=heads, o=o, ow=ow, n=n),
        grid=(n // bi, nj),
        in_specs=[
            pl.BlockSpec((bi, h2), lambda i, j: (i, 0)),
            pl.BlockSpec((h2, bj), lambda i, j: (0, j)),
            pl.BlockSpec((bi, bj), lambda i, j: (i, j)),
            pl.BlockSpec((n + npad, heads * ow), lambda i, j: (0, 0)),
            pl.BlockSpec((1, ho), lambda i, j: (0, 0)),
        ],
        out_specs=pl.BlockSpec((bi, ho), lambda i, j: (i, 0)),
        out_shape=jax.ShapeDtypeStruct((n, ho), jnp.float32),
        scratch_shapes=[
            pltpu.VMEM((bi, heads * ow), jnp.float32),
        ],
        compiler_params=pltpu.CompilerParams(
            dimension_semantics=("parallel", "arbitrary")),
    )(src_e, dst_et, adj, waug, bias)


def _pack_w(w):
    h, f, o = w.shape
    return jnp.transpose(w, (1, 0, 2)).reshape(f, h * o)


def _pack_a(a_src, a_dst):
    h, o = a_src.shape
    z = jnp.zeros((h * o, 2 * h), jnp.float32)
    for i in range(h):
        z = z.at[i * o:(i + 1) * o, i].set(a_src[i])
        z = z.at[i * o:(i + 1) * o, h + i].set(a_dst[i])
    return z


def _block_sizes(n):
    if n % 2000 == 0:
        return 2000, 1000, 2048
    return n, n, n


def kernel(x, adj, W1, a_src1, a_dst1, b1, W2, a_src2, a_dst2, b2):
    heads = W1.shape[0]
    bn, bi, bj = _block_sizes(adj.shape[0])
    o1, o2 = W1.shape[2], W2.shape[2]
    ow1 = 128 * (-(-(o1 + 1) // 128))
    ow2 = 128 * (-(-(o2 + 1) // 128))

    waug1, e1 = _project(x, _pack_w(W1), _pack_a(a_src1, a_dst1), bn, ow1)
    h1 = _gat_layer(waug1, e1, adj, b1.reshape(1, -1), heads, o1, ow1, bi, bj)

    waug2, e2 = _project(h1, _pack_w(W2), _pack_a(a_src2, a_dst2), bn, ow2)
    return _gat_layer(waug2, e2, adj, b2.reshape(1, -1), heads, o2, ow2, bi, bj)
